# Initial kernel scaffold; baseline (speedup 1.0000x reference)
import jax
import jax.numpy as jnp
from jax.experimental import pallas as pl


def kernel(target_word, context_word, negative_example, W_target, W_context):
    raise NotImplementedError("write your pallas kernel here")


# speedup vs baseline: 4.6612x; 4.6612x over previous
import functools

import jax
import jax.numpy as jnp
from jax import lax
from jax.experimental import pallas as pl
from jax.experimental.pallas import tpu as pltpu
from jax.experimental.pallas import tpu_sc as plsc

NC = 2
NS = 16
NW = NC * NS
L = 16

CB = 16
NQ = 4


def _tree_sum(vs):
    while len(vs) > 1:
        vs = [vs[i] + vs[i + 1] for i in range(0, len(vs) - 1, 2)] + (
            [vs[-1]] if len(vs) % 2 else []
        )
    return vs[0]


def _sc_scores_kernel(B, NEG, D):
    assert D == NQ * L
    assert B % (NW * CB) == 0
    b_per_w = B // NW
    n_chunks = b_per_w // CB
    neg_per_chunk = CB * NEG
    neg_dma_rows = 80
    n_neg_dma = neg_per_chunk // neg_dma_rows

    mesh = plsc.VectorSubcoreMesh(
        core_axis_name="c", subcore_axis_name="s", num_cores=NC, num_subcores=NS
    )

    @functools.partial(
        pl.kernel,
        out_type=(
            jax.ShapeDtypeStruct((B,), jnp.float32),
            jax.ShapeDtypeStruct((B,), jnp.float32),
        ),
        mesh=mesh,
        compiler_params=pltpu.CompilerParams(
            needs_layout_passes=False, use_tc_tiling_on_sc=False
        ),
        scratch_types=dict(
            t_idx=pltpu.VMEM((CB,), jnp.int32),
            c_idx=pltpu.VMEM((CB,), jnp.int32),
            n_idx=pltpu.VMEM((n_neg_dma, neg_dma_rows), jnp.int32),
            t_rows=pltpu.VMEM((CB, D), jnp.float32),
            c_rows=pltpu.VMEM((CB, D), jnp.float32),
            n_rows=pltpu.VMEM((neg_per_chunk, D), jnp.float32),
            pos_v=pltpu.VMEM((b_per_w,), jnp.float32),
            neg_v=pltpu.VMEM((b_per_w,), jnp.float32),
            sem=pltpu.SemaphoreType.DMA,
        ),
    )
    def sc_kernel(
        tgt_hbm, ctx_hbm, negidx_hbm, wt_hbm, wc_hbm,
        pos_hbm, negdot_hbm,
        t_idx, c_idx, n_idx, t_rows, c_rows, n_rows, pos_v, neg_v, sem,
    ):
        wid = lax.axis_index("s") * NC + lax.axis_index("c")
        base = wid * b_per_w
        iota = lax.iota(jnp.int32, L)

        def chunk_body(c, carry):
            cb = base + c * CB
            pltpu.sync_copy(tgt_hbm.at[pl.ds(cb, CB)], t_idx)
            pltpu.sync_copy(ctx_hbm.at[pl.ds(cb, CB)], c_idx)
            for j in range(n_neg_dma):
                pltpu.sync_copy(
                    negidx_hbm.at[pl.ds(cb * NEG + j * neg_dma_rows, neg_dma_rows)],
                    n_idx.at[j],
                )

            cps = [
                pltpu.async_copy(wt_hbm.at[t_idx], t_rows, sem),
                pltpu.async_copy(wc_hbm.at[c_idx], c_rows, sem),
            ]
            for j in range(n_neg_dma):
                cps.append(
                    pltpu.async_copy(
                        wc_hbm.at[n_idx.at[j]],
                        n_rows.at[pl.ds(j * neg_dma_rows, neg_dma_rows)],
                        sem,
                    )
                )
            for cp in cps:
                cp.wait()

            pos_acc = jnp.zeros((L,), jnp.float32)
            neg_acc = jnp.zeros((L,), jnp.float32)
            for j in range(CB):
                t_q = [t_rows[j, pl.ds(q * L, L)] for q in range(NQ)]
                c_q = [c_rows[j, pl.ds(q * L, L)] for q in range(NQ)]
                pos_s = jnp.sum(_tree_sum([t_q[q] * c_q[q] for q in range(NQ)]))
                n_q = [
                    _tree_sum([n_rows[j * NEG + k, pl.ds(q * L, L)] for k in range(NEG)])
                    for q in range(NQ)
                ]
                neg_s = jnp.sum(_tree_sum([t_q[q] * n_q[q] for q in range(NQ)]))
                lane = j % L
                pos_acc = jnp.where(iota == lane, pos_s, pos_acc)
                neg_acc = jnp.where(iota == lane, neg_s, neg_acc)
                if lane == L - 1:
                    o = c * CB + (j // L) * L
                    pos_v[pl.ds(o, L)] = pos_acc
                    neg_v[pl.ds(o, L)] = neg_acc
            return carry

        lax.fori_loop(0, n_chunks, chunk_body, 0)
        pltpu.sync_copy(pos_v, pos_hbm.at[pl.ds(base, b_per_w)])
        pltpu.sync_copy(neg_v, negdot_hbm.at[pl.ds(base, b_per_w)])

    return sc_kernel


def _tc_loss_kernel(pos_ref, neg_ref, out_ref):
    p = pos_ref[...]
    n = neg_ref[...]

    def ls(x):
        return jnp.minimum(x, 0.0) - jnp.log1p(jnp.exp(-jnp.abs(x)))

    out_ref[0, 0] = -(jnp.sum(ls(p)) + jnp.sum(ls(-n)))


def kernel(target_word, context_word, negative_example, W_target, W_context):
    B = target_word.shape[0]
    NEG = negative_example.shape[1]
    D = W_target.shape[1]

    tgt = target_word.astype(jnp.int32)
    ctx = context_word.astype(jnp.int32)
    neg_flat = negative_example.astype(jnp.int32).reshape(B * NEG)

    sc = _sc_scores_kernel(B, NEG, D)
    pos_dot, neg_dot = sc(tgt, ctx, neg_flat, W_target, W_context)

    r = B // 128
    loss = pl.pallas_call(
        _tc_loss_kernel,
        out_shape=jax.ShapeDtypeStruct((1, 1), jnp.float32),
        out_specs=pl.BlockSpec(memory_space=pltpu.SMEM),
    )(pos_dot.reshape(r, 128), neg_dot.reshape(r, 128))
    return loss[0, 0]
